# Initial kernel scaffold; baseline (speedup 1.0000x reference)
import jax
import jax.numpy as jnp
from jax.experimental import pallas as pl


def kernel(x, edge_index, edge_attr, batch, global_feat, cluster, Wn, bn_, We, be, gat_W, gat_att, gat_bias, gat_bn_g, gat_bn_b, bn_g, bn_b, W1, b1, W2, b2):
    raise NotImplementedError("write your pallas kernel here")


# speedup vs baseline: 7.5879x; 7.5879x over previous
import functools

import jax
import jax.numpy as jnp
from jax import lax
from jax.experimental import pallas as pl
from jax.experimental.pallas import tpu as pltpu

N_NODES = 10000
N_EDGES = 160000
NEURONS = 64
HEADS = 4
NL = 3
BE = 3200
NB = N_EDGES // BE


def _softplus(x):
    return jnp.maximum(x, 0.0) + jnp.log1p(jnp.exp(-jnp.abs(x)))


def _k0a_body(x_ref, wn_ref, bn_ref, out_ref):
    out_ref[...] = jnp.dot(x_ref[...], wn_ref[...],
                           preferred_element_type=jnp.float32) + bn_ref[...]


def _node_embed(x, Wn, bn_):
    return pl.pallas_call(
        _k0a_body,
        out_shape=jax.ShapeDtypeStruct((N_NODES, NEURONS), jnp.float32),
    )(x, Wn, bn_.reshape(1, NEURONS))


def _k0b_body(ea_ref, we_ref, be_ref, out_ref):
    t = jnp.dot(ea_ref[...], we_ref[...],
                preferred_element_type=jnp.float32) + be_ref[...]
    out_ref[...] = jnp.where(t >= 0, t, 0.2 * t)


def _edge_embed(edge_attr, We, be):
    return pl.pallas_call(
        _k0b_body,
        grid=(NB,),
        in_specs=[
            pl.BlockSpec((BE, 41), lambda b: (b, 0)),
            pl.BlockSpec((41, NEURONS), lambda b: (0, 0)),
            pl.BlockSpec((1, NEURONS), lambda b: (0, 0)),
        ],
        out_specs=pl.BlockSpec((BE, NEURONS), lambda b: (b, 0)),
        out_shape=jax.ShapeDtypeStruct((N_EDGES, NEURONS), jnp.float32),
    )(edge_attr, We, be.reshape(1, NEURONS))


def _eye4():
    r = lax.broadcasted_iota(jnp.int32, (HEADS, HEADS), 0)
    c = lax.broadcasted_iota(jnp.int32, (HEADS, HEADS), 1)
    return (r == c).astype(jnp.float32)


def _ab_body(hi_ref, hj_ref, ea_ref, wt_ref, wb_ref, ai_ref, aj_ref,
             g_ref, b_ref, et_ref, sp_scr, sum_scr, sq_scr, mx_scr):
    p = pl.program_id(0)
    b = pl.program_id(1)

    @pl.when(p == 0)
    def _phase0():
        @pl.when(b == 0)
        def _init():
            sum_scr[...] = jnp.zeros((HEADS, 1), jnp.float32)
            sq_scr[...] = jnp.zeros((HEADS, 1), jnp.float32)
            mx_scr[...] = jnp.full((HEADS, 1), -1e30, jnp.float32)

        r = jnp.dot(ea_ref[...], wb_ref[...], preferred_element_type=jnp.float32)
        xi = _softplus(jnp.dot(hi_ref[...], wt_ref[...],
                               preferred_element_type=jnp.float32) + r)
        xj = _softplus(jnp.dot(hj_ref[...], wt_ref[...],
                               preferred_element_type=jnp.float32) + r)
        pi = xi * ai_ref[...]
        pj = xj * aj_ref[...]
        cols = []
        for h in range(HEADS):
            s = (jnp.sum(pi[:, h * NEURONS:(h + 1) * NEURONS], axis=1, keepdims=True)
                 + jnp.sum(pj[:, h * NEURONS:(h + 1) * NEURONS], axis=1, keepdims=True))
            cols.append(s)
        logit = jnp.concatenate(cols, axis=1)
        logit_t = jax.lax.dot_general(_eye4(), logit, (((1,), (1,)), ((), ())),
                                      preferred_element_type=jnp.float32)
        sp = _softplus(logit_t)
        sp_scr[0:HEADS, pl.ds(b * BE, BE)] = sp
        sum_scr[...] += jnp.sum(sp, axis=1, keepdims=True)
        sq_scr[...] += jnp.sum(sp * sp, axis=1, keepdims=True)
        mx_scr[...] = jnp.maximum(mx_scr[...],
                                  jnp.max(sp, axis=1, keepdims=True))

    @pl.when(p == 1)
    def _phase1():
        n = jnp.float32(N_EDGES)
        m = sum_scr[...] / n
        v = sq_scr[...] / n - m * m
        inv = g_ref[...] * lax.rsqrt(v + 1e-5)
        gmax = _softplus((mx_scr[...] - m) * inv + b_ref[...])
        sp = sp_scr[0:HEADS, pl.ds(b * BE, BE)]
        at = _softplus((sp - m) * inv + b_ref[...])
        et_ref[0:HEADS, :] = jnp.exp(at - gmax)
        et_ref[HEADS:8, :] = jnp.zeros((8 - HEADS, BE), jnp.float32)


def _edge_attention(HI, HJ, EA, Wt, Wb, att_i, att_j, g, bb):
    zero = lambda p, b: (0, 0)
    ph0 = lambda p, b: (jnp.where(p == 0, b, 0), 0)
    ph1 = lambda p, b: (0, jnp.where(p == 0, 0, b))
    return pl.pallas_call(
        _ab_body,
        grid=(2, NB),
        in_specs=[
            pl.BlockSpec((BE, NEURONS), ph0),
            pl.BlockSpec((BE, NEURONS), ph0),
            pl.BlockSpec((BE, NEURONS), ph0),
            pl.BlockSpec((NEURONS, HEADS * NEURONS), zero),
            pl.BlockSpec((NEURONS, HEADS * NEURONS), zero),
            pl.BlockSpec((1, HEADS * NEURONS), zero),
            pl.BlockSpec((1, HEADS * NEURONS), zero),
            pl.BlockSpec((HEADS, 1), zero),
            pl.BlockSpec((HEADS, 1), zero),
        ],
        out_specs=pl.BlockSpec((8, BE), ph1),
        out_shape=jax.ShapeDtypeStruct((8, N_EDGES), jnp.float32),
        scratch_shapes=[
            pltpu.VMEM((8, N_EDGES), jnp.float32),
            pltpu.VMEM((HEADS, 1), jnp.float32),
            pltpu.VMEM((HEADS, 1), jnp.float32),
            pltpu.VMEM((HEADS, 1), jnp.float32),
        ],
    )(HI, HJ, EA, Wt, Wb, att_i, att_j, g, bb)


def _c_body(hj_ref, ea_ref, et_ref, sr_ref, wt_ref, wb_ref, out_ref):
    r = jnp.dot(ea_ref[...], wb_ref[...], preferred_element_type=jnp.float32)
    xj = _softplus(jnp.dot(hj_ref[...], wt_ref[...],
                           preferred_element_type=jnp.float32) + r)
    w_t = et_ref[0:HEADS, :] / (sr_ref[0:HEADS, :] + 1e-16)
    w = jax.lax.dot_general(w_t, _eye4(), (((0,), (0,)), ((), ())),
                            preferred_element_type=jnp.float32)
    acc = jnp.zeros((hj_ref.shape[0], NEURONS), jnp.float32)
    for h in range(HEADS):
        acc += xj[:, h * NEURONS:(h + 1) * NEURONS] * w[:, h:h + 1]
    out_ref[...] = acc


def _edge_messages(HJ, EA, ET, SR, Wt, Wb):
    zero = lambda b: (0, 0)
    blk = lambda b: (b, 0)
    blk_t = lambda b: (0, b)
    return pl.pallas_call(
        _c_body,
        grid=(NB,),
        in_specs=[
            pl.BlockSpec((BE, NEURONS), blk),
            pl.BlockSpec((BE, NEURONS), blk),
            pl.BlockSpec((8, BE), blk_t),
            pl.BlockSpec((8, BE), blk_t),
            pl.BlockSpec((NEURONS, HEADS * NEURONS), zero),
            pl.BlockSpec((NEURONS, HEADS * NEURONS), zero),
        ],
        out_specs=pl.BlockSpec((BE, NEURONS), blk),
        out_shape=jax.ShapeDtypeStruct((N_EDGES, NEURONS), jnp.float32),
    )(HJ, EA, ET, SR, Wt, Wb)


def _n_body(agg_ref, bias_ref, g_ref, b_ref, out_ref):
    h1 = (agg_ref[0:N_NODES, :] + agg_ref[N_NODES:2 * N_NODES, :]) / HEADS \
        + bias_ref[...]
    m = jnp.mean(h1, axis=0, keepdims=True)
    d = h1 - m
    v = jnp.mean(d * d, axis=0, keepdims=True)
    out_ref[...] = _softplus(d * lax.rsqrt(v + 1e-5) * g_ref[...] + b_ref[...])


def _node_update(aggr2, bias, g, bb):
    return pl.pallas_call(
        _n_body,
        out_shape=jax.ShapeDtypeStruct((N_NODES, NEURONS), jnp.float32),
    )(aggr2, bias.reshape(1, NEURONS), g.reshape(1, NEURONS),
      bb.reshape(1, NEURONS))


def _r_body(h_ref, batch_ref, gf_ref, w1h_ref, w1g_ref, b1_ref, w2_ref,
            b2_ref, out_ref):
    h = h_ref[...]
    bvec = batch_ref[...]
    cols = lax.broadcasted_iota(jnp.int32, (1, 64), 1)
    onehot = (bvec == cols).astype(jnp.float32)
    gw = jnp.dot(gf_ref[...], w1g_ref[...],
                 preferred_element_type=jnp.float32)
    a = _softplus(jnp.dot(h, w1h_ref[...], preferred_element_type=jnp.float32)
                  + jnp.dot(onehot, gw, preferred_element_type=jnp.float32)
                  + b1_ref[...])
    a = jnp.dot(a, w2_ref[...], preferred_element_type=jnp.float32) \
        + b2_ref[...]
    amax = jnp.max(a)
    e = jnp.exp(a - amax)
    sb = jax.lax.dot_general(onehot, e, (((0,), (0,)), ((), ())))
    srow = jnp.dot(onehot, sb, preferred_element_type=jnp.float32)
    w = e / (srow + 1e-16)
    hw = h * w
    ynum = jax.lax.dot_general(onehot, hw, (((0,), (0,)), ((), ())))
    counts = jnp.sum(onehot, axis=0, keepdims=True)
    out_ref[...] = ynum / jnp.maximum(counts, 1.0).reshape(64, 1)


def _readout(h, batch, global_feat, W1, b1, W2, b2):
    return pl.pallas_call(
        _r_body,
        out_shape=jax.ShapeDtypeStruct((64, NEURONS), jnp.float32),
    )(h, batch.reshape(N_NODES, 1), global_feat,
      W1[:NEURONS], W1[NEURONS:], b1.reshape(1, 32), W2, b2.reshape(1, 1))


def _gather_rows(table, idx):
    return table[idx]


def _scatter_add(values, idx, num):
    return jax.ops.segment_sum(values, idx, num_segments=num)


def kernel(x, edge_index, edge_attr, batch, global_feat, cluster, Wn, bn_,
           We, be, gat_W, gat_att, gat_bias, gat_bn_g, gat_bn_b, bn_g, bn_b,
           W1, b1, W2, b2):
    row = edge_index[0]
    col = edge_index[1]

    h = _node_embed(x, Wn, bn_)
    ea = _edge_embed(edge_attr, We, be)

    for l in range(NL):
        Wt = gat_W[l][:NEURONS]
        Wb = gat_W[l][NEURONS:]
        att = gat_att[l]
        att_i = att[:, :NEURONS].reshape(1, HEADS * NEURONS)
        att_j = att[:, NEURONS:].reshape(1, HEADS * NEURONS)

        HI = _gather_rows(h, row)
        HJ = _gather_rows(h, col)
        ET = _edge_attention(HI, HJ, ea, Wt, Wb, att_i, att_j,
                             gat_bn_g[l].reshape(HEADS, 1),
                             gat_bn_b[l].reshape(HEADS, 1))
        S = _scatter_add(ET[:HEADS].T, row, N_NODES)
        SR = jnp.concatenate([_gather_rows(S, row).T,
                              jnp.zeros((8 - HEADS, N_EDGES), jnp.float32)], 0)
        C = _edge_messages(HJ, ea, ET, SR, Wt, Wb)
        aggr = _scatter_add(C, row, N_NODES)
        aggr2 = jnp.concatenate([aggr, jnp.zeros_like(aggr)], axis=0)
        h = _node_update(aggr2, gat_bias[l], bn_g[l], bn_b[l])

    return _readout(h, batch, global_feat, W1, b1, W2, b2)


# speedup vs baseline: 8.1938x; 1.0798x over previous
import functools

import jax
import jax.numpy as jnp
from jax import lax
from jax.experimental import pallas as pl
from jax.experimental.pallas import tpu as pltpu
from jax.experimental.pallas import tpu_sc as plsc

N_NODES = 10000
N_EDGES = 160000
NEURONS = 64
HEADS = 4
NL = 3
BE = 3200
NB = N_EDGES // BE


def _softplus(x):
    return jnp.maximum(x, 0.0) + jnp.log1p(jnp.exp(-jnp.abs(x)))


def _k0a_body(x_ref, wn_ref, bn_ref, out_ref):
    h = jnp.dot(x_ref[...], wn_ref[...],
                preferred_element_type=jnp.float32) + bn_ref[...]
    out_ref[...] = jnp.concatenate(
        [h, jnp.zeros((N_NODES, 128 - NEURONS), jnp.float32)], axis=1)


def _node_embed(x, Wn, bn_):
    return pl.pallas_call(
        _k0a_body,
        out_shape=jax.ShapeDtypeStruct((N_NODES, 128), jnp.float32),
    )(x, Wn, bn_.reshape(1, NEURONS))


def _k0b_body(ea_ref, we_ref, be_ref, out_ref):
    t = jnp.dot(ea_ref[...], we_ref[...],
                preferred_element_type=jnp.float32) + be_ref[...]
    out_ref[...] = jnp.where(t >= 0, t, 0.2 * t)


def _edge_embed(edge_attr, We, be):
    return pl.pallas_call(
        _k0b_body,
        grid=(NB,),
        in_specs=[
            pl.BlockSpec((BE, 41), lambda b: (b, 0)),
            pl.BlockSpec((41, NEURONS), lambda b: (0, 0)),
            pl.BlockSpec((1, NEURONS), lambda b: (0, 0)),
        ],
        out_specs=pl.BlockSpec((BE, NEURONS), lambda b: (b, 0)),
        out_shape=jax.ShapeDtypeStruct((N_EDGES, NEURONS), jnp.float32),
    )(edge_attr, We, be.reshape(1, NEURONS))


def _eye4():
    r = lax.broadcasted_iota(jnp.int32, (HEADS, HEADS), 0)
    c = lax.broadcasted_iota(jnp.int32, (HEADS, HEADS), 1)
    return (r == c).astype(jnp.float32)


def _ab_body(hi_ref, hj_ref, ea_ref, wt_ref, wb_ref, ai_ref, aj_ref,
             g_ref, b_ref, et_ref, sp_scr, sum_scr, sq_scr, mx_scr):
    p = pl.program_id(0)
    b = pl.program_id(1)

    @pl.when(p == 0)
    def _phase0():
        @pl.when(b == 0)
        def _init():
            sum_scr[...] = jnp.zeros((HEADS, 1), jnp.float32)
            sq_scr[...] = jnp.zeros((HEADS, 1), jnp.float32)
            mx_scr[...] = jnp.full((HEADS, 1), -1e30, jnp.float32)

        r = jnp.dot(ea_ref[...], wb_ref[...], preferred_element_type=jnp.float32)
        xi = _softplus(jnp.dot(hi_ref[:, :NEURONS], wt_ref[...],
                               preferred_element_type=jnp.float32) + r)
        xj = _softplus(jnp.dot(hj_ref[:, :NEURONS], wt_ref[...],
                               preferred_element_type=jnp.float32) + r)
        pi = xi * ai_ref[...]
        pj = xj * aj_ref[...]
        cols = []
        for h in range(HEADS):
            s = (jnp.sum(pi[:, h * NEURONS:(h + 1) * NEURONS], axis=1, keepdims=True)
                 + jnp.sum(pj[:, h * NEURONS:(h + 1) * NEURONS], axis=1, keepdims=True))
            cols.append(s)
        logit = jnp.concatenate(cols, axis=1)
        logit_t = jax.lax.dot_general(_eye4(), logit, (((1,), (1,)), ((), ())),
                                      preferred_element_type=jnp.float32)
        sp = _softplus(logit_t)
        sp_scr[0:HEADS, pl.ds(b * BE, BE)] = sp
        sum_scr[...] += jnp.sum(sp, axis=1, keepdims=True)
        sq_scr[...] += jnp.sum(sp * sp, axis=1, keepdims=True)
        mx_scr[...] = jnp.maximum(mx_scr[...],
                                  jnp.max(sp, axis=1, keepdims=True))

    @pl.when(p == 1)
    def _phase1():
        n = jnp.float32(N_EDGES)
        m = sum_scr[...] / n
        v = sq_scr[...] / n - m * m
        inv = g_ref[...] * lax.rsqrt(v + 1e-5)
        gmax = _softplus((mx_scr[...] - m) * inv + b_ref[...])
        sp = sp_scr[0:HEADS, pl.ds(b * BE, BE)]
        at = _softplus((sp - m) * inv + b_ref[...])
        et_ref[0:HEADS, :] = jnp.exp(at - gmax)
        et_ref[HEADS:8, :] = jnp.zeros((8 - HEADS, BE), jnp.float32)


def _edge_attention(HI, HJ, EA, Wt, Wb, att_i, att_j, g, bb):
    zero = lambda p, b: (0, 0)
    ph0 = lambda p, b: (jnp.where(p == 0, b, 0), 0)
    ph1 = lambda p, b: (0, jnp.where(p == 0, 0, b))
    return pl.pallas_call(
        _ab_body,
        grid=(2, NB),
        in_specs=[
            pl.BlockSpec((BE, 128), ph0),
            pl.BlockSpec((BE, 128), ph0),
            pl.BlockSpec((BE, NEURONS), ph0),
            pl.BlockSpec((NEURONS, HEADS * NEURONS), zero),
            pl.BlockSpec((NEURONS, HEADS * NEURONS), zero),
            pl.BlockSpec((1, HEADS * NEURONS), zero),
            pl.BlockSpec((1, HEADS * NEURONS), zero),
            pl.BlockSpec((HEADS, 1), zero),
            pl.BlockSpec((HEADS, 1), zero),
        ],
        out_specs=pl.BlockSpec((8, BE), ph1),
        out_shape=jax.ShapeDtypeStruct((8, N_EDGES), jnp.float32),
        scratch_shapes=[
            pltpu.VMEM((8, N_EDGES), jnp.float32),
            pltpu.VMEM((HEADS, 1), jnp.float32),
            pltpu.VMEM((HEADS, 1), jnp.float32),
            pltpu.VMEM((HEADS, 1), jnp.float32),
        ],
    )(HI, HJ, EA, Wt, Wb, att_i, att_j, g, bb)


def _c_body(hj_ref, ea_ref, et_ref, sr_ref, wt_ref, wb_ref, out_ref):
    r = jnp.dot(ea_ref[...], wb_ref[...], preferred_element_type=jnp.float32)
    xj = _softplus(jnp.dot(hj_ref[:, :NEURONS], wt_ref[...],
                           preferred_element_type=jnp.float32) + r)
    w_t = et_ref[0:HEADS, :] / (sr_ref[0:HEADS, :] + 1e-16)
    w = jax.lax.dot_general(w_t, _eye4(), (((0,), (0,)), ((), ())),
                            preferred_element_type=jnp.float32)
    acc = jnp.zeros((hj_ref.shape[0], NEURONS), jnp.float32)
    for h in range(HEADS):
        acc += xj[:, h * NEURONS:(h + 1) * NEURONS] * w[:, h:h + 1]
    out_ref[...] = jnp.concatenate(
        [acc, jnp.zeros((hj_ref.shape[0], 128 - NEURONS), jnp.float32)], 1)


def _edge_messages(HJ, EA, ET, SR, Wt, Wb):
    zero = lambda b: (0, 0)
    blk = lambda b: (b, 0)
    blk_t = lambda b: (0, b)
    return pl.pallas_call(
        _c_body,
        grid=(NB,),
        in_specs=[
            pl.BlockSpec((BE, 128), blk),
            pl.BlockSpec((BE, NEURONS), blk),
            pl.BlockSpec((8, BE), blk_t),
            pl.BlockSpec((8, BE), blk_t),
            pl.BlockSpec((NEURONS, HEADS * NEURONS), zero),
            pl.BlockSpec((NEURONS, HEADS * NEURONS), zero),
        ],
        out_specs=pl.BlockSpec((BE, 128), blk),
        out_shape=jax.ShapeDtypeStruct((N_EDGES, 128), jnp.float32),
    )(HJ, EA, ET, SR, Wt, Wb)


def _n_body(agg_ref, bias_ref, g_ref, b_ref, out_ref):
    h1 = (agg_ref[0:N_NODES, :NEURONS]
          + agg_ref[NP:NP + N_NODES, :NEURONS]) / HEADS + bias_ref[...]
    m = jnp.mean(h1, axis=0, keepdims=True)
    d = h1 - m
    v = jnp.mean(d * d, axis=0, keepdims=True)
    out = _softplus(d * lax.rsqrt(v + 1e-5) * g_ref[...] + b_ref[...])
    out_ref[...] = jnp.concatenate(
        [out, jnp.zeros((N_NODES, 128 - NEURONS), jnp.float32)], 1)


def _node_update(aggr2, bias, g, bb):
    return pl.pallas_call(
        _n_body,
        out_shape=jax.ShapeDtypeStruct((N_NODES, 128), jnp.float32),
    )(aggr2, bias.reshape(1, NEURONS), g.reshape(1, NEURONS),
      bb.reshape(1, NEURONS))


def _r_body(h_ref, batch_ref, gf_ref, w1h_ref, w1g_ref, b1_ref, w2_ref,
            b2_ref, out_ref):
    h = h_ref[:, :NEURONS]
    bvec = batch_ref[...]
    cols = lax.broadcasted_iota(jnp.int32, (1, 64), 1)
    onehot = (bvec == cols).astype(jnp.float32)
    gw = jnp.dot(gf_ref[...], w1g_ref[...],
                 preferred_element_type=jnp.float32)
    a = _softplus(jnp.dot(h, w1h_ref[...], preferred_element_type=jnp.float32)
                  + jnp.dot(onehot, gw, preferred_element_type=jnp.float32)
                  + b1_ref[...])
    a = jnp.dot(a, w2_ref[...], preferred_element_type=jnp.float32) \
        + b2_ref[...]
    amax = jnp.max(a)
    e = jnp.exp(a - amax)
    sb = jax.lax.dot_general(onehot, e, (((0,), (0,)), ((), ())))
    srow = jnp.dot(onehot, sb, preferred_element_type=jnp.float32)
    w = e / (srow + 1e-16)
    hw = h * w
    ynum = jax.lax.dot_general(onehot, hw, (((0,), (0,)), ((), ())))
    counts = jnp.sum(onehot, axis=0, keepdims=True)
    out_ref[...] = ynum / jnp.maximum(counts, 1.0).reshape(64, 1)


def _readout(h, batch, global_feat, W1, b1, W2, b2):
    return pl.pallas_call(
        _r_body,
        out_shape=jax.ShapeDtypeStruct((64, NEURONS), jnp.float32),
    )(h, batch.reshape(N_NODES, 1), global_feat,
      W1[:NEURONS], W1[NEURONS:], b1.reshape(1, 32), W2, b2.reshape(1, 1))


NC = 2
NS = 16
NW = NC * NS
CH = 40
NP = 10240


def _sc_mesh():
    return plsc.VectorSubcoreMesh(core_axis_name="c", subcore_axis_name="s")


def _gather_pair(h, row, col):
    per_w = N_EDGES // NW
    nch = per_w // CH

    @functools.partial(
        pl.kernel,
        out_type=(jax.ShapeDtypeStruct((N_EDGES, 128), jnp.float32),
                  jax.ShapeDtypeStruct((N_EDGES, 128), jnp.float32)),
        mesh=_sc_mesh(),
        scratch_types=[
            pltpu.VMEM((CH,), jnp.int32),
            pltpu.VMEM((CH, 128), jnp.float32),
            pltpu.SemaphoreType.DMA,
        ],
    )
    def k(h_hbm, row_hbm, col_hbm, hi_hbm, hj_hbm, idx_v, rows_v, sem):
        wid = lax.axis_index("s") * NC + lax.axis_index("c")
        base = wid * per_w

        def chunk(i, idx_hbm, out_hbm):
            off = base + i * CH
            pltpu.sync_copy(idx_hbm.at[pl.ds(off, CH)], idx_v)
            pltpu.async_copy(h_hbm.at[idx_v], rows_v, sem).wait()
            pltpu.sync_copy(rows_v, out_hbm.at[pl.ds(off, CH)])

        def body1(i, c):
            chunk(i, row_hbm, hi_hbm)
            return c

        def body2(i, c):
            chunk(i, col_hbm, hj_hbm)
            return c

        lax.fori_loop(0, nch, body1, 0)
        lax.fori_loop(0, nch, body2, 0)

    return k(h, row, col)


def _attn_norm(et0, et1, et2, et3, row, zeros_n):
    per_t = N_EDGES // NS
    nch = per_t // CH
    rows_per_tile = NP // NS

    out_t = tuple(jax.ShapeDtypeStruct((N_EDGES,), jnp.float32)
                  for _ in range(HEADS))

    @functools.partial(
        pl.kernel,
        out_type=out_t,
        mesh=_sc_mesh(),
        scratch_types=[
            [pltpu.VMEM_SHARED((NP,), jnp.float32) for _ in range(HEADS)],
            pltpu.VMEM((CH,), jnp.int32),
            pltpu.VMEM((CH,), jnp.float32),
            pltpu.SemaphoreType.DMA,
        ],
    )
    def k(e0, e1, e2, e3, row_hbm, z_hbm, s0, s1, s2, s3,
          shead, idx_v, val_v, sem):
        cid = lax.axis_index("c")
        sid = lax.axis_index("s")
        ins = [e0, e1, e2, e3]
        outs = [s0, s1, s2, s3]

        @pl.when(cid == 0)
        def _go():
            zoff = sid * rows_per_tile
            for hh in range(HEADS):
                pltpu.sync_copy(z_hbm.at[pl.ds(zoff, rows_per_tile)],
                                shead[hh].at[pl.ds(zoff, rows_per_tile)])
            plsc.subcore_barrier()

            base = sid * per_t

            def scat(i, c):
                off = base + i * CH
                pltpu.sync_copy(row_hbm.at[pl.ds(off, CH)], idx_v)
                for hh in range(HEADS):
                    pltpu.sync_copy(ins[hh].at[pl.ds(off, CH)], val_v)
                    pltpu.sync_copy(val_v, shead[hh].at[idx_v], add=True)
                return c

            lax.fori_loop(0, nch, scat, 0)
            plsc.subcore_barrier()

            def gath(i, c):
                off = base + i * CH
                pltpu.sync_copy(row_hbm.at[pl.ds(off, CH)], idx_v)
                for hh in range(HEADS):
                    pltpu.async_copy(shead[hh].at[idx_v], val_v, sem).wait()
                    pltpu.sync_copy(val_v, outs[hh].at[pl.ds(off, CH)])
                return c

            lax.fori_loop(0, nch, gath, 0)

    return k(et0, et1, et2, et3, row, zeros_n)


def _scatter_rows(values, row, zeros_tab):
    per_w = N_EDGES // NW
    nch = per_w // CH
    rows_per_tile = NP // NS

    @functools.partial(
        pl.kernel,
        out_type=jax.ShapeDtypeStruct((2 * NP, 128), jnp.float32),
        mesh=_sc_mesh(),
        scratch_types=[
            pltpu.VMEM_SHARED((NP, 128), jnp.float32),
            pltpu.VMEM((CH,), jnp.int32),
            pltpu.VMEM((CH, 128), jnp.float32),
        ],
    )
    def k(val_hbm, row_hbm, z_hbm, out_hbm, shared, idx_v, rows_v):
        cid = lax.axis_index("c")
        sid = lax.axis_index("s")
        wid = sid * NC + cid
        zoff = sid * rows_per_tile
        pltpu.sync_copy(z_hbm.at[pl.ds(zoff, rows_per_tile)],
                        shared.at[pl.ds(zoff, rows_per_tile)])
        plsc.subcore_barrier()

        base = wid * per_w

        def scat(i, c):
            off = base + i * CH
            pltpu.sync_copy(row_hbm.at[pl.ds(off, CH)], idx_v)
            pltpu.sync_copy(val_hbm.at[pl.ds(off, CH)], rows_v)
            pltpu.sync_copy(rows_v, shared.at[idx_v], add=True)
            return c

        lax.fori_loop(0, nch, scat, 0)
        plsc.subcore_barrier()
        pltpu.sync_copy(shared.at[pl.ds(zoff, rows_per_tile)],
                        out_hbm.at[pl.ds(cid * NP + zoff, rows_per_tile)])

    return k(values, row, zeros_tab)


def kernel(x, edge_index, edge_attr, batch, global_feat, cluster, Wn, bn_,
           We, be, gat_W, gat_att, gat_bias, gat_bn_g, gat_bn_b, bn_g, bn_b,
           W1, b1, W2, b2):
    row = edge_index[0]
    col = edge_index[1]
    zeros_n = jnp.zeros((NP,), jnp.float32)
    zeros_tab = jnp.zeros((NP, 128), jnp.float32)

    h = _node_embed(x, Wn, bn_)
    ea = _edge_embed(edge_attr, We, be)

    for l in range(NL):
        Wt = gat_W[l][:NEURONS]
        Wb = gat_W[l][NEURONS:]
        att = gat_att[l]
        att_i = att[:, :NEURONS].reshape(1, HEADS * NEURONS)
        att_j = att[:, NEURONS:].reshape(1, HEADS * NEURONS)

        HI, HJ = _gather_pair(h, row, col)
        ET = _edge_attention(HI, HJ, ea, Wt, Wb, att_i, att_j,
                             gat_bn_g[l].reshape(HEADS, 1),
                             gat_bn_b[l].reshape(HEADS, 1))
        srs = _attn_norm(ET[0], ET[1], ET[2], ET[3], row, zeros_n)
        SR = jnp.concatenate(
            [s.reshape(1, N_EDGES) for s in srs]
            + [jnp.zeros((8 - HEADS, N_EDGES), jnp.float32)], 0)
        C = _edge_messages(HJ, ea, ET, SR, Wt, Wb)
        aggr2 = _scatter_rows(C, row, zeros_tab)
        h = _node_update(aggr2, gat_bias[l], bn_g[l], bn_b[l])

    return _readout(h, batch, global_feat, W1, b1, W2, b2)


# speedup vs baseline: 12.4645x; 1.5212x over previous
import functools

import jax
import jax.numpy as jnp
from jax import lax
from jax.experimental import pallas as pl
from jax.experimental.pallas import tpu as pltpu
from jax.experimental.pallas import tpu_sc as plsc

N_NODES = 10000
N_EDGES = 160000
NEURONS = 64
HEADS = 4
NL = 3
BE = 3200
NB = N_EDGES // BE


def _softplus(x):
    return jnp.maximum(x, 0.0) + jnp.log1p(jnp.exp(-jnp.abs(x)))


def _k0a_body(x_ref, wn_ref, bn_ref, out_ref):
    h = jnp.dot(x_ref[...], wn_ref[...],
                preferred_element_type=jnp.float32) + bn_ref[...]
    out_ref[...] = jnp.concatenate(
        [h, jnp.zeros((N_NODES, 128 - NEURONS), jnp.float32)], axis=1)


def _node_embed(x, Wn, bn_):
    return pl.pallas_call(
        _k0a_body,
        out_shape=jax.ShapeDtypeStruct((N_NODES, 128), jnp.float32),
    )(x, Wn, bn_.reshape(1, NEURONS))


def _k0b_body(ea_ref, we_ref, be_ref, out_ref):
    t = jnp.dot(ea_ref[...], we_ref[...],
                preferred_element_type=jnp.float32) + be_ref[...]
    out_ref[...] = jnp.where(t >= 0, t, 0.2 * t)


def _edge_embed(edge_attr, We, be):
    return pl.pallas_call(
        _k0b_body,
        grid=(NB,),
        in_specs=[
            pl.BlockSpec((BE, 41), lambda b: (b, 0)),
            pl.BlockSpec((41, NEURONS), lambda b: (0, 0)),
            pl.BlockSpec((1, NEURONS), lambda b: (0, 0)),
        ],
        out_specs=pl.BlockSpec((BE, NEURONS), lambda b: (b, 0)),
        out_shape=jax.ShapeDtypeStruct((N_EDGES, NEURONS), jnp.float32),
    )(edge_attr, We, be.reshape(1, NEURONS))


def _eye4():
    r = lax.broadcasted_iota(jnp.int32, (HEADS, HEADS), 0)
    c = lax.broadcasted_iota(jnp.int32, (HEADS, HEADS), 1)
    return (r == c).astype(jnp.float32)


def _ab_body(hi_ref, hj_ref, ea_ref, wt_ref, wb_ref, ai_ref, aj_ref,
             g_ref, b_ref, et_ref, sp_scr, sum_scr, sq_scr, mx_scr):
    p = pl.program_id(0)
    b = pl.program_id(1)

    @pl.when(p == 0)
    def _phase0():
        @pl.when(b == 0)
        def _init():
            sum_scr[...] = jnp.zeros((HEADS, 1), jnp.float32)
            sq_scr[...] = jnp.zeros((HEADS, 1), jnp.float32)
            mx_scr[...] = jnp.full((HEADS, 1), -1e30, jnp.float32)

        r = jnp.dot(ea_ref[...], wb_ref[...], preferred_element_type=jnp.float32)
        xi = _softplus(jnp.dot(hi_ref[:, :NEURONS], wt_ref[...],
                               preferred_element_type=jnp.float32) + r)
        xj = _softplus(jnp.dot(hj_ref[:, :NEURONS], wt_ref[...],
                               preferred_element_type=jnp.float32) + r)
        pi = xi * ai_ref[...]
        pj = xj * aj_ref[...]
        cols = []
        for h in range(HEADS):
            s = (jnp.sum(pi[:, h * NEURONS:(h + 1) * NEURONS], axis=1, keepdims=True)
                 + jnp.sum(pj[:, h * NEURONS:(h + 1) * NEURONS], axis=1, keepdims=True))
            cols.append(s)
        logit = jnp.concatenate(cols, axis=1)
        logit_t = jax.lax.dot_general(_eye4(), logit, (((1,), (1,)), ((), ())),
                                      preferred_element_type=jnp.float32)
        sp = _softplus(logit_t)
        sp_scr[0:HEADS, pl.ds(b * BE, BE)] = sp
        sum_scr[...] += jnp.sum(sp, axis=1, keepdims=True)
        sq_scr[...] += jnp.sum(sp * sp, axis=1, keepdims=True)
        mx_scr[...] = jnp.maximum(mx_scr[...],
                                  jnp.max(sp, axis=1, keepdims=True))

    @pl.when(p == 1)
    def _phase1():
        n = jnp.float32(N_EDGES)
        m = sum_scr[...] / n
        v = sq_scr[...] / n - m * m
        inv = g_ref[...] * lax.rsqrt(v + 1e-5)
        gmax = _softplus((mx_scr[...] - m) * inv + b_ref[...])
        sp = sp_scr[0:HEADS, pl.ds(b * BE, BE)]
        at = _softplus((sp - m) * inv + b_ref[...])
        et_ref[0:HEADS, :] = jnp.exp(at - gmax)
        et_ref[HEADS:8, :] = jnp.zeros((8 - HEADS, BE), jnp.float32)


def _edge_attention(HI, HJ, EA, Wt, Wb, att_i, att_j, g, bb):
    zero = lambda p, b: (0, 0)
    ph0 = lambda p, b: (jnp.where(p == 0, b, 0), 0)
    ph1 = lambda p, b: (0, jnp.where(p == 0, 0, b))
    return pl.pallas_call(
        _ab_body,
        grid=(2, NB),
        in_specs=[
            pl.BlockSpec((BE, 128), ph0),
            pl.BlockSpec((BE, 128), ph0),
            pl.BlockSpec((BE, NEURONS), ph0),
            pl.BlockSpec((NEURONS, HEADS * NEURONS), zero),
            pl.BlockSpec((NEURONS, HEADS * NEURONS), zero),
            pl.BlockSpec((1, HEADS * NEURONS), zero),
            pl.BlockSpec((1, HEADS * NEURONS), zero),
            pl.BlockSpec((HEADS, 1), zero),
            pl.BlockSpec((HEADS, 1), zero),
        ],
        out_specs=pl.BlockSpec((8, BE), ph1),
        out_shape=jax.ShapeDtypeStruct((8, N_EDGES), jnp.float32),
        scratch_shapes=[
            pltpu.VMEM((8, N_EDGES), jnp.float32),
            pltpu.VMEM((HEADS, 1), jnp.float32),
            pltpu.VMEM((HEADS, 1), jnp.float32),
            pltpu.VMEM((HEADS, 1), jnp.float32),
        ],
    )(HI, HJ, EA, Wt, Wb, att_i, att_j, g, bb)


def _c_body(hj_ref, ea_ref, et_ref, sr_ref, wt_ref, wb_ref, out_ref):
    r = jnp.dot(ea_ref[...], wb_ref[...], preferred_element_type=jnp.float32)
    xj = _softplus(jnp.dot(hj_ref[:, :NEURONS], wt_ref[...],
                           preferred_element_type=jnp.float32) + r)
    w_t = et_ref[0:HEADS, :] / (sr_ref[0:HEADS, :] + 1e-16)
    w = jax.lax.dot_general(w_t, _eye4(), (((0,), (0,)), ((), ())),
                            preferred_element_type=jnp.float32)
    acc = jnp.zeros((hj_ref.shape[0], NEURONS), jnp.float32)
    for h in range(HEADS):
        acc += xj[:, h * NEURONS:(h + 1) * NEURONS] * w[:, h:h + 1]
    out_ref[...] = jnp.concatenate(
        [acc, jnp.zeros((hj_ref.shape[0], 128 - NEURONS), jnp.float32)], 1)


def _edge_messages(HJ, EA, ET, SR, Wt, Wb):
    zero = lambda b: (0, 0)
    blk = lambda b: (b, 0)
    blk_t = lambda b: (0, b)
    return pl.pallas_call(
        _c_body,
        grid=(NB,),
        in_specs=[
            pl.BlockSpec((BE, 128), blk),
            pl.BlockSpec((BE, NEURONS), blk),
            pl.BlockSpec((8, BE), blk_t),
            pl.BlockSpec((8, BE), blk_t),
            pl.BlockSpec((NEURONS, HEADS * NEURONS), zero),
            pl.BlockSpec((NEURONS, HEADS * NEURONS), zero),
        ],
        out_specs=pl.BlockSpec((BE, 128), blk),
        out_shape=jax.ShapeDtypeStruct((N_EDGES, 128), jnp.float32),
    )(HJ, EA, ET, SR, Wt, Wb)


def _n_body(agg_ref, bias_ref, g_ref, b_ref, out_ref):
    h1 = (agg_ref[0:N_NODES, :NEURONS]
          + agg_ref[NP:NP + N_NODES, :NEURONS]) / HEADS + bias_ref[...]
    m = jnp.mean(h1, axis=0, keepdims=True)
    d = h1 - m
    v = jnp.mean(d * d, axis=0, keepdims=True)
    out = _softplus(d * lax.rsqrt(v + 1e-5) * g_ref[...] + b_ref[...])
    out_ref[...] = jnp.concatenate(
        [out, jnp.zeros((N_NODES, 128 - NEURONS), jnp.float32)], 1)


def _node_update(aggr2, bias, g, bb):
    return pl.pallas_call(
        _n_body,
        out_shape=jax.ShapeDtypeStruct((N_NODES, 128), jnp.float32),
    )(aggr2, bias.reshape(1, NEURONS), g.reshape(1, NEURONS),
      bb.reshape(1, NEURONS))


def _r_body(h_ref, batch_ref, gf_ref, w1h_ref, w1g_ref, b1_ref, w2_ref,
            b2_ref, out_ref):
    h = h_ref[:, :NEURONS]
    bvec = batch_ref[...]
    cols = lax.broadcasted_iota(jnp.int32, (1, 64), 1)
    onehot = (bvec == cols).astype(jnp.float32)
    gw = jnp.dot(gf_ref[...], w1g_ref[...],
                 preferred_element_type=jnp.float32)
    a = _softplus(jnp.dot(h, w1h_ref[...], preferred_element_type=jnp.float32)
                  + jnp.dot(onehot, gw, preferred_element_type=jnp.float32)
                  + b1_ref[...])
    a = jnp.dot(a, w2_ref[...], preferred_element_type=jnp.float32) \
        + b2_ref[...]
    amax = jnp.max(a)
    e = jnp.exp(a - amax)
    sb = jax.lax.dot_general(onehot, e, (((0,), (0,)), ((), ())))
    srow = jnp.dot(onehot, sb, preferred_element_type=jnp.float32)
    w = e / (srow + 1e-16)
    hw = h * w
    ynum = jax.lax.dot_general(onehot, hw, (((0,), (0,)), ((), ())))
    counts = jnp.sum(onehot, axis=0, keepdims=True)
    out_ref[...] = ynum / jnp.maximum(counts, 1.0).reshape(64, 1)


def _readout(h, batch, global_feat, W1, b1, W2, b2):
    return pl.pallas_call(
        _r_body,
        out_shape=jax.ShapeDtypeStruct((64, NEURONS), jnp.float32),
    )(h, batch.reshape(N_NODES, 1), global_feat,
      W1[:NEURONS], W1[NEURONS:], b1.reshape(1, 32), W2, b2.reshape(1, 1))


NC = 2
NS = 16
NW = NC * NS
CH = 128
NCH = N_EDGES // CH
NP = 10240


def _chunk_range(wid, total, nworkers):
    base = total // nworkers
    extra = total % nworkers
    start = wid * base + jnp.minimum(wid, extra)
    n = base + jnp.where(wid < extra, 1, 0)
    return start, n


def _sc_mesh():
    return plsc.VectorSubcoreMesh(core_axis_name="c", subcore_axis_name="s")


def _gather_pair(h, row, col):
    @functools.partial(
        pl.kernel,
        out_type=(jax.ShapeDtypeStruct((N_EDGES, 128), jnp.float32),
                  jax.ShapeDtypeStruct((N_EDGES, 128), jnp.float32)),
        mesh=_sc_mesh(),
        scratch_types=[
            pltpu.VMEM((CH,), jnp.int32),
            pltpu.VMEM((CH, 128), jnp.float32),
            pltpu.SemaphoreType.DMA,
        ],
    )
    def k(h_hbm, row_hbm, col_hbm, hi_hbm, hj_hbm, idx_v, rows_v, sem):
        wid = lax.axis_index("s") * NC + lax.axis_index("c")
        start, n = _chunk_range(wid, NCH, NW)

        def chunk(i, idx_hbm, out_hbm):
            off = (start + i) * CH
            pltpu.sync_copy(idx_hbm.at[pl.ds(off, CH)], idx_v)
            pltpu.async_copy(h_hbm.at[idx_v], rows_v, sem).wait()
            pltpu.sync_copy(rows_v, out_hbm.at[pl.ds(off, CH)])

        def body1(i, c):
            chunk(i, row_hbm, hi_hbm)
            return c

        def body2(i, c):
            chunk(i, col_hbm, hj_hbm)
            return c

        lax.fori_loop(0, n, body1, 0)
        lax.fori_loop(0, n, body2, 0)

    return k(h, row, col)


def _attn_norm(et0, et1, et2, et3, row, zeros_n):
    rows_per_tile = NP // NS

    out_t = tuple(jax.ShapeDtypeStruct((N_EDGES,), jnp.float32)
                  for _ in range(HEADS))

    @functools.partial(
        pl.kernel,
        out_type=out_t,
        mesh=_sc_mesh(),
        scratch_types=[
            [pltpu.VMEM_SHARED((NP,), jnp.float32) for _ in range(HEADS)],
            pltpu.VMEM((CH,), jnp.int32),
            pltpu.VMEM((CH,), jnp.float32),
            pltpu.SemaphoreType.DMA,
        ],
    )
    def k(e0, e1, e2, e3, row_hbm, z_hbm, s0, s1, s2, s3,
          shead, idx_v, val_v, sem):
        cid = lax.axis_index("c")
        sid = lax.axis_index("s")
        ins = [e0, e1, e2, e3]
        outs = [s0, s1, s2, s3]

        @pl.when(cid == 0)
        def _go():
            zoff = sid * rows_per_tile
            for hh in range(HEADS):
                pltpu.sync_copy(z_hbm.at[pl.ds(zoff, rows_per_tile)],
                                shead[hh].at[pl.ds(zoff, rows_per_tile)])
            plsc.subcore_barrier()

            start, n = _chunk_range(sid, NCH, NS)

            def scat(i, c):
                off = (start + i) * CH
                pltpu.sync_copy(row_hbm.at[pl.ds(off, CH)], idx_v)
                for hh in range(HEADS):
                    pltpu.sync_copy(ins[hh].at[pl.ds(off, CH)], val_v)
                    pltpu.sync_copy(val_v, shead[hh].at[idx_v], add=True)
                return c

            lax.fori_loop(0, n, scat, 0)
            plsc.subcore_barrier()

            def gath(i, c):
                off = (start + i) * CH
                pltpu.sync_copy(row_hbm.at[pl.ds(off, CH)], idx_v)
                for hh in range(HEADS):
                    pltpu.async_copy(shead[hh].at[idx_v], val_v, sem).wait()
                    pltpu.sync_copy(val_v, outs[hh].at[pl.ds(off, CH)])
                return c

            lax.fori_loop(0, n, gath, 0)

    return k(et0, et1, et2, et3, row, zeros_n)


def _scatter_rows(values, row, zeros_tab):
    rows_per_tile = NP // NS

    @functools.partial(
        pl.kernel,
        out_type=jax.ShapeDtypeStruct((2 * NP, 128), jnp.float32),
        mesh=_sc_mesh(),
        scratch_types=[
            pltpu.VMEM_SHARED((NP, 128), jnp.float32),
            pltpu.VMEM((CH,), jnp.int32),
            pltpu.VMEM((CH, 128), jnp.float32),
        ],
    )
    def k(val_hbm, row_hbm, z_hbm, out_hbm, shared, idx_v, rows_v):
        cid = lax.axis_index("c")
        sid = lax.axis_index("s")
        wid = sid * NC + cid
        zoff = sid * rows_per_tile
        pltpu.sync_copy(z_hbm.at[pl.ds(zoff, rows_per_tile)],
                        shared.at[pl.ds(zoff, rows_per_tile)])
        plsc.subcore_barrier()

        start, n = _chunk_range(wid, NCH, NW)

        def scat(i, c):
            off = (start + i) * CH
            pltpu.sync_copy(row_hbm.at[pl.ds(off, CH)], idx_v)
            pltpu.sync_copy(val_hbm.at[pl.ds(off, CH)], rows_v)
            pltpu.sync_copy(rows_v, shared.at[idx_v], add=True)
            return c

        lax.fori_loop(0, n, scat, 0)
        plsc.subcore_barrier()
        pltpu.sync_copy(shared.at[pl.ds(zoff, rows_per_tile)],
                        out_hbm.at[pl.ds(cid * NP + zoff, rows_per_tile)])

    return k(values, row, zeros_tab)


def kernel(x, edge_index, edge_attr, batch, global_feat, cluster, Wn, bn_,
           We, be, gat_W, gat_att, gat_bias, gat_bn_g, gat_bn_b, bn_g, bn_b,
           W1, b1, W2, b2):
    row = edge_index[0]
    col = edge_index[1]
    zeros_n = jnp.zeros((NP,), jnp.float32)
    zeros_tab = jnp.zeros((NP, 128), jnp.float32)

    h = _node_embed(x, Wn, bn_)
    ea = _edge_embed(edge_attr, We, be)

    for l in range(NL):
        Wt = gat_W[l][:NEURONS]
        Wb = gat_W[l][NEURONS:]
        att = gat_att[l]
        att_i = att[:, :NEURONS].reshape(1, HEADS * NEURONS)
        att_j = att[:, NEURONS:].reshape(1, HEADS * NEURONS)

        HI, HJ = _gather_pair(h, row, col)
        ET = _edge_attention(HI, HJ, ea, Wt, Wb, att_i, att_j,
                             gat_bn_g[l].reshape(HEADS, 1),
                             gat_bn_b[l].reshape(HEADS, 1))
        srs = _attn_norm(ET[0], ET[1], ET[2], ET[3], row, zeros_n)
        SR = jnp.concatenate(
            [s.reshape(1, N_EDGES) for s in srs]
            + [jnp.zeros((8 - HEADS, N_EDGES), jnp.float32)], 0)
        C = _edge_messages(HJ, ea, ET, SR, Wt, Wb)
        aggr2 = _scatter_rows(C, row, zeros_tab)
        h = _node_update(aggr2, gat_bias[l], bn_g[l], bn_b[l])

    return _readout(h, batch, global_feat, W1, b1, W2, b2)


# speedup vs baseline: 12.8604x; 1.0318x over previous
import functools

import jax
import jax.numpy as jnp
from jax import lax
from jax.experimental import pallas as pl
from jax.experimental.pallas import tpu as pltpu
from jax.experimental.pallas import tpu_sc as plsc

N_NODES = 10000
N_EDGES = 160000
NEURONS = 64
HEADS = 4
NL = 3
BE = 3200
NB = N_EDGES // BE


def _softplus(x):
    return jnp.maximum(x, 0.0) + jnp.log1p(jnp.exp(-jnp.abs(x)))


def _k0a_body(x_ref, wn_ref, bn_ref, out_ref):
    h = jnp.dot(x_ref[...], wn_ref[...],
                preferred_element_type=jnp.float32) + bn_ref[...]
    out_ref[...] = jnp.concatenate(
        [h, jnp.zeros((N_NODES, 128 - NEURONS), jnp.float32)], axis=1)


def _node_embed(x, Wn, bn_):
    return pl.pallas_call(
        _k0a_body,
        out_shape=jax.ShapeDtypeStruct((N_NODES, 128), jnp.float32),
    )(x, Wn, bn_.reshape(1, NEURONS))


def _k0b_body(ea_ref, we_ref, be_ref, out_ref):
    t = jnp.dot(ea_ref[...], we_ref[...],
                preferred_element_type=jnp.float32) + be_ref[...]
    out_ref[...] = jnp.where(t >= 0, t, 0.2 * t)


def _edge_embed(edge_attr, We, be):
    return pl.pallas_call(
        _k0b_body,
        grid=(NB,),
        in_specs=[
            pl.BlockSpec((BE, 41), lambda b: (b, 0)),
            pl.BlockSpec((41, NEURONS), lambda b: (0, 0)),
            pl.BlockSpec((1, NEURONS), lambda b: (0, 0)),
        ],
        out_specs=pl.BlockSpec((BE, NEURONS), lambda b: (b, 0)),
        out_shape=jax.ShapeDtypeStruct((N_EDGES, NEURONS), jnp.float32),
    )(edge_attr, We, be.reshape(1, NEURONS))


def _eye4():
    r = lax.broadcasted_iota(jnp.int32, (HEADS, HEADS), 0)
    c = lax.broadcasted_iota(jnp.int32, (HEADS, HEADS), 1)
    return (r == c).astype(jnp.float32)


def _ab_body(hi_ref, hj_ref, ea_ref, wt_ref, wb_ref, ai_ref, aj_ref,
             g_ref, b_ref, et_ref, sp_scr, sum_scr, sq_scr, mx_scr):
    p = pl.program_id(0)
    b = pl.program_id(1)

    @pl.when(p == 0)
    def _phase0():
        @pl.when(b == 0)
        def _init():
            sum_scr[...] = jnp.zeros((HEADS, 1), jnp.float32)
            sq_scr[...] = jnp.zeros((HEADS, 1), jnp.float32)
            mx_scr[...] = jnp.full((HEADS, 1), -1e30, jnp.float32)

        r = jnp.dot(ea_ref[...], wb_ref[...], preferred_element_type=jnp.float32)
        xi = _softplus(jnp.dot(hi_ref[:, :NEURONS], wt_ref[...],
                               preferred_element_type=jnp.float32) + r)
        xj = _softplus(jnp.dot(hj_ref[:, :NEURONS], wt_ref[...],
                               preferred_element_type=jnp.float32) + r)
        pi = xi * ai_ref[...]
        pj = xj * aj_ref[...]
        cols = []
        for h in range(HEADS):
            s = (jnp.sum(pi[:, h * NEURONS:(h + 1) * NEURONS], axis=1, keepdims=True)
                 + jnp.sum(pj[:, h * NEURONS:(h + 1) * NEURONS], axis=1, keepdims=True))
            cols.append(s)
        logit = jnp.concatenate(cols, axis=1)
        logit_t = jax.lax.dot_general(_eye4(), logit, (((1,), (1,)), ((), ())),
                                      preferred_element_type=jnp.float32)
        sp = _softplus(logit_t)
        sp_scr[0:HEADS, pl.ds(b * BE, BE)] = sp
        sum_scr[...] += jnp.sum(sp, axis=1, keepdims=True)
        sq_scr[...] += jnp.sum(sp * sp, axis=1, keepdims=True)
        mx_scr[...] = jnp.maximum(mx_scr[...],
                                  jnp.max(sp, axis=1, keepdims=True))

    @pl.when(p == 1)
    def _phase1():
        n = jnp.float32(N_EDGES)
        m = sum_scr[...] / n
        v = sq_scr[...] / n - m * m
        inv = g_ref[...] * lax.rsqrt(v + 1e-5)
        gmax = _softplus((mx_scr[...] - m) * inv + b_ref[...])
        sp = sp_scr[0:HEADS, pl.ds(b * BE, BE)]
        at = _softplus((sp - m) * inv + b_ref[...])
        et_ref[0:HEADS, :] = jnp.exp(at - gmax)
        et_ref[HEADS:8, :] = jnp.zeros((8 - HEADS, BE), jnp.float32)


def _edge_attention(HI, HJ, EA, Wt, Wb, att_i, att_j, g, bb):
    zero = lambda p, b: (0, 0)
    ph0 = lambda p, b: (jnp.where(p == 0, b, 0), 0)
    ph1 = lambda p, b: (0, jnp.where(p == 0, 0, b))
    return pl.pallas_call(
        _ab_body,
        grid=(2, NB),
        in_specs=[
            pl.BlockSpec((BE, 128), ph0),
            pl.BlockSpec((BE, 128), ph0),
            pl.BlockSpec((BE, NEURONS), ph0),
            pl.BlockSpec((NEURONS, HEADS * NEURONS), zero),
            pl.BlockSpec((NEURONS, HEADS * NEURONS), zero),
            pl.BlockSpec((1, HEADS * NEURONS), zero),
            pl.BlockSpec((1, HEADS * NEURONS), zero),
            pl.BlockSpec((HEADS, 1), zero),
            pl.BlockSpec((HEADS, 1), zero),
        ],
        out_specs=pl.BlockSpec((8, BE), ph1),
        out_shape=jax.ShapeDtypeStruct((8, N_EDGES), jnp.float32),
        scratch_shapes=[
            pltpu.VMEM((8, N_EDGES), jnp.float32),
            pltpu.VMEM((HEADS, 1), jnp.float32),
            pltpu.VMEM((HEADS, 1), jnp.float32),
            pltpu.VMEM((HEADS, 1), jnp.float32),
        ],
    )(HI, HJ, EA, Wt, Wb, att_i, att_j, g, bb)


def _c_body(hj_ref, ea_ref, et_ref, sr_ref, wt_ref, wb_ref, out_ref):
    r = jnp.dot(ea_ref[...], wb_ref[...], preferred_element_type=jnp.float32)
    xj = _softplus(jnp.dot(hj_ref[:, :NEURONS], wt_ref[...],
                           preferred_element_type=jnp.float32) + r)
    w_t = et_ref[0:HEADS, :] / (sr_ref[0:HEADS, :] + 1e-16)
    w = jax.lax.dot_general(w_t, _eye4(), (((0,), (0,)), ((), ())),
                            preferred_element_type=jnp.float32)
    acc = jnp.zeros((hj_ref.shape[0], NEURONS), jnp.float32)
    for h in range(HEADS):
        acc += xj[:, h * NEURONS:(h + 1) * NEURONS] * w[:, h:h + 1]
    out_ref[...] = jnp.concatenate(
        [acc, jnp.zeros((hj_ref.shape[0], 128 - NEURONS), jnp.float32)], 1)


def _edge_messages(HJ, EA, ET, SR, Wt, Wb):
    zero = lambda b: (0, 0)
    blk = lambda b: (b, 0)
    blk_t = lambda b: (0, b)
    return pl.pallas_call(
        _c_body,
        grid=(NB,),
        in_specs=[
            pl.BlockSpec((BE, 128), blk),
            pl.BlockSpec((BE, NEURONS), blk),
            pl.BlockSpec((8, BE), blk_t),
            pl.BlockSpec((8, BE), blk_t),
            pl.BlockSpec((NEURONS, HEADS * NEURONS), zero),
            pl.BlockSpec((NEURONS, HEADS * NEURONS), zero),
        ],
        out_specs=pl.BlockSpec((BE, 128), blk),
        out_shape=jax.ShapeDtypeStruct((N_EDGES, 128), jnp.float32),
    )(HJ, EA, ET, SR, Wt, Wb)


def _n_body(agg_ref, bias_ref, g_ref, b_ref, out_ref):
    h1 = (agg_ref[0:N_NODES, :NEURONS]
          + agg_ref[NP:NP + N_NODES, :NEURONS]) / HEADS + bias_ref[...]
    m = jnp.mean(h1, axis=0, keepdims=True)
    d = h1 - m
    v = jnp.mean(d * d, axis=0, keepdims=True)
    out = _softplus(d * lax.rsqrt(v + 1e-5) * g_ref[...] + b_ref[...])
    out_ref[...] = jnp.concatenate(
        [out, jnp.zeros((N_NODES, 128 - NEURONS), jnp.float32)], 1)


def _node_update(aggr2, bias, g, bb):
    return pl.pallas_call(
        _n_body,
        out_shape=jax.ShapeDtypeStruct((N_NODES, 128), jnp.float32),
    )(aggr2, bias.reshape(1, NEURONS), g.reshape(1, NEURONS),
      bb.reshape(1, NEURONS))


def _r_body(h_ref, batch_ref, gf_ref, w1h_ref, w1g_ref, b1_ref, w2_ref,
            b2_ref, out_ref):
    h = h_ref[:, :NEURONS]
    bvec = batch_ref[...]
    cols = lax.broadcasted_iota(jnp.int32, (1, 64), 1)
    onehot = (bvec == cols).astype(jnp.float32)
    gw = jnp.dot(gf_ref[...], w1g_ref[...],
                 preferred_element_type=jnp.float32)
    a = _softplus(jnp.dot(h, w1h_ref[...], preferred_element_type=jnp.float32)
                  + jnp.dot(onehot, gw, preferred_element_type=jnp.float32)
                  + b1_ref[...])
    a = jnp.dot(a, w2_ref[...], preferred_element_type=jnp.float32) \
        + b2_ref[...]
    amax = jnp.max(a)
    e = jnp.exp(a - amax)
    sb = jax.lax.dot_general(onehot, e, (((0,), (0,)), ((), ())))
    srow = jnp.dot(onehot, sb, preferred_element_type=jnp.float32)
    w = e / (srow + 1e-16)
    hw = h * w
    ynum = jax.lax.dot_general(onehot, hw, (((0,), (0,)), ((), ())))
    counts = jnp.sum(onehot, axis=0, keepdims=True)
    out_ref[...] = ynum / jnp.maximum(counts, 1.0).reshape(64, 1)


def _readout(h, batch, global_feat, W1, b1, W2, b2):
    return pl.pallas_call(
        _r_body,
        out_shape=jax.ShapeDtypeStruct((64, NEURONS), jnp.float32),
    )(h, batch.reshape(N_NODES, 1), global_feat,
      W1[:NEURONS], W1[NEURONS:], b1.reshape(1, 32), W2, b2.reshape(1, 1))


NC = 2
NS = 16
NW = NC * NS
CH = 128
NCH = N_EDGES // CH
NP = 10240


def _chunk_range(wid, total, nworkers):
    base = total // nworkers
    extra = total % nworkers
    start = wid * base + jnp.minimum(wid, extra)
    n = base + jnp.where(wid < extra, 1, 0)
    return start, n


def _sc_mesh():
    return plsc.VectorSubcoreMesh(core_axis_name="c", subcore_axis_name="s")


MAXW = NCH // NW + 1
NBUF = 4


def _gather_pair(h, row, col):
    @functools.partial(
        pl.kernel,
        out_type=(jax.ShapeDtypeStruct((N_EDGES, 128), jnp.float32),
                  jax.ShapeDtypeStruct((N_EDGES, 128), jnp.float32)),
        mesh=_sc_mesh(),
        scratch_types=[
            [pltpu.VMEM((CH,), jnp.int32) for _ in range(NBUF)],
            [pltpu.VMEM((CH, 128), jnp.float32) for _ in range(NBUF)],
            [pltpu.SemaphoreType.DMA for _ in range(NBUF)],
        ],
    )
    def k(h_hbm, row_hbm, col_hbm, hi_hbm, hj_hbm, idxb, bufs, semg):
        wid = lax.axis_index("s") * NC + lax.axis_index("c")
        start, n = _chunk_range(wid, NCH, NW)
        n4 = n // NBUF
        rem = n - NBUF * n4

        def do_array(idx_hbm, out_hbm):
            def gather_k(base, k_):
                for b in range(k_):
                    pltpu.sync_copy(
                        idx_hbm.at[pl.ds((start + base + b) * CH, CH)],
                        idxb[b])
                cps = [pltpu.async_copy(
                    h_hbm.at[idxb[b]], bufs[b], semg[b]) for b in range(k_)]
                for b in range(k_):
                    cps[b].wait()
                    pltpu.sync_copy(
                        bufs[b],
                        out_hbm.at[pl.ds((start + base + b) * CH, CH)])

            def grp(g, c):
                gather_k(g * NBUF, NBUF)
                return c

            lax.fori_loop(0, n4, grp, 0)

            @pl.when(rem == 3)
            def _tail():
                gather_k(n4 * NBUF, 3)

        do_array(row_hbm, hi_hbm)
        do_array(col_hbm, hj_hbm)

    return k(h, row, col)


def _attn_norm(et0, et1, et2, et3, row, zeros_n):
    rows_per_tile = NP // NS

    out_t = tuple(jax.ShapeDtypeStruct((N_EDGES,), jnp.float32)
                  for _ in range(HEADS))

    @functools.partial(
        pl.kernel,
        out_type=out_t,
        mesh=_sc_mesh(),
        scratch_types=[
            [pltpu.VMEM_SHARED((NP,), jnp.float32) for _ in range(HEADS)],
            pltpu.VMEM((CH,), jnp.int32),
            pltpu.VMEM((CH,), jnp.float32),
            pltpu.SemaphoreType.DMA,
        ],
    )
    def k(e0, e1, e2, e3, row_hbm, z_hbm, s0, s1, s2, s3,
          shead, idx_v, val_v, sem):
        cid = lax.axis_index("c")
        sid = lax.axis_index("s")
        ins = [e0, e1, e2, e3]
        outs = [s0, s1, s2, s3]

        @pl.when(cid == 0)
        def _go():
            zoff = sid * rows_per_tile
            for hh in range(HEADS):
                pltpu.sync_copy(z_hbm.at[pl.ds(zoff, rows_per_tile)],
                                shead[hh].at[pl.ds(zoff, rows_per_tile)])
            plsc.subcore_barrier()

            start, n = _chunk_range(sid, NCH, NS)

            def scat(i, c):
                off = (start + i) * CH
                pltpu.sync_copy(row_hbm.at[pl.ds(off, CH)], idx_v)
                for hh in range(HEADS):
                    pltpu.sync_copy(ins[hh].at[pl.ds(off, CH)], val_v)
                    pltpu.sync_copy(val_v, shead[hh].at[idx_v], add=True)
                return c

            lax.fori_loop(0, n, scat, 0)
            plsc.subcore_barrier()

            def gath(i, c):
                off = (start + i) * CH
                pltpu.sync_copy(row_hbm.at[pl.ds(off, CH)], idx_v)
                for hh in range(HEADS):
                    pltpu.async_copy(shead[hh].at[idx_v], val_v, sem).wait()
                    pltpu.sync_copy(val_v, outs[hh].at[pl.ds(off, CH)])
                return c

            lax.fori_loop(0, n, gath, 0)

    return k(et0, et1, et2, et3, row, zeros_n)


def _scatter_rows(values, row, zeros_tab):
    rows_per_tile = NP // NS

    @functools.partial(
        pl.kernel,
        out_type=jax.ShapeDtypeStruct((2 * NP, 128), jnp.float32),
        mesh=_sc_mesh(),
        scratch_types=[
            pltpu.VMEM_SHARED((NP, 128), jnp.float32),
            pltpu.VMEM((CH,), jnp.int32),
            pltpu.VMEM((CH, 128), jnp.float32),
        ],
    )
    def k(val_hbm, row_hbm, z_hbm, out_hbm, shared, idx_v, rows_v):
        cid = lax.axis_index("c")
        sid = lax.axis_index("s")
        wid = sid * NC + cid
        zoff = sid * rows_per_tile
        pltpu.sync_copy(z_hbm.at[pl.ds(zoff, rows_per_tile)],
                        shared.at[pl.ds(zoff, rows_per_tile)])
        plsc.subcore_barrier()

        start, n = _chunk_range(wid, NCH, NW)

        def scat(i, c):
            off = (start + i) * CH
            pltpu.sync_copy(row_hbm.at[pl.ds(off, CH)], idx_v)
            pltpu.sync_copy(val_hbm.at[pl.ds(off, CH)], rows_v)
            pltpu.sync_copy(rows_v, shared.at[idx_v], add=True)
            return c

        lax.fori_loop(0, n, scat, 0)
        plsc.subcore_barrier()
        pltpu.sync_copy(shared.at[pl.ds(zoff, rows_per_tile)],
                        out_hbm.at[pl.ds(cid * NP + zoff, rows_per_tile)])

    return k(values, row, zeros_tab)


def kernel(x, edge_index, edge_attr, batch, global_feat, cluster, Wn, bn_,
           We, be, gat_W, gat_att, gat_bias, gat_bn_g, gat_bn_b, bn_g, bn_b,
           W1, b1, W2, b2):
    pad = jnp.zeros((MAXW * CH,), jnp.int32)
    row = jnp.concatenate([edge_index[0], pad])
    col = jnp.concatenate([edge_index[1], pad])
    zeros_n = jnp.zeros((NP,), jnp.float32)
    zeros_tab = jnp.zeros((NP, 128), jnp.float32)

    h = _node_embed(x, Wn, bn_)
    ea = _edge_embed(edge_attr, We, be)

    for l in range(NL):
        Wt = gat_W[l][:NEURONS]
        Wb = gat_W[l][NEURONS:]
        att = gat_att[l]
        att_i = att[:, :NEURONS].reshape(1, HEADS * NEURONS)
        att_j = att[:, NEURONS:].reshape(1, HEADS * NEURONS)

        HI, HJ = _gather_pair(h, row, col)
        ET = _edge_attention(HI, HJ, ea, Wt, Wb, att_i, att_j,
                             gat_bn_g[l].reshape(HEADS, 1),
                             gat_bn_b[l].reshape(HEADS, 1))
        srs = _attn_norm(ET[0], ET[1], ET[2], ET[3], row, zeros_n)
        SR = jnp.concatenate(
            [s.reshape(1, N_EDGES) for s in srs]
            + [jnp.zeros((8 - HEADS, N_EDGES), jnp.float32)], 0)
        C = _edge_messages(HJ, ea, ET, SR, Wt, Wb)
        aggr2 = _scatter_rows(C, row, zeros_tab)
        h = _node_update(aggr2, gat_bias[l], bn_g[l], bn_b[l])

    return _readout(h, batch, global_feat, W1, b1, W2, b2)


# speedup vs baseline: 13.9895x; 1.0878x over previous
import functools

import jax
import jax.numpy as jnp
from jax import lax
from jax.experimental import pallas as pl
from jax.experimental.pallas import tpu as pltpu
from jax.experimental.pallas import tpu_sc as plsc

N_NODES = 10000
N_EDGES = 160000
NEURONS = 64
HEADS = 4
NL = 3
BE = 3200
NB = N_EDGES // BE


def _softplus(x):
    return jnp.maximum(x, 0.0) + jnp.log1p(jnp.exp(-jnp.abs(x)))


def _k0a_body(x_ref, wn_ref, bn_ref, out_ref):
    h = jnp.dot(x_ref[...], wn_ref[...],
                preferred_element_type=jnp.float32) + bn_ref[...]
    out_ref[...] = jnp.concatenate(
        [h, jnp.zeros((N_NODES, 128 - NEURONS), jnp.float32)], axis=1)


def _node_embed(x, Wn, bn_):
    return pl.pallas_call(
        _k0a_body,
        out_shape=jax.ShapeDtypeStruct((N_NODES, 128), jnp.float32),
    )(x, Wn, bn_.reshape(1, NEURONS))


def _k0b_body(ea_ref, we_ref, be_ref, out_ref):
    t = jnp.dot(ea_ref[...], we_ref[...],
                preferred_element_type=jnp.float32) + be_ref[...]
    out_ref[...] = jnp.where(t >= 0, t, 0.2 * t)


def _edge_embed(edge_attr, We, be):
    return pl.pallas_call(
        _k0b_body,
        grid=(NB,),
        in_specs=[
            pl.BlockSpec((BE, 41), lambda b: (b, 0)),
            pl.BlockSpec((41, NEURONS), lambda b: (0, 0)),
            pl.BlockSpec((1, NEURONS), lambda b: (0, 0)),
        ],
        out_specs=pl.BlockSpec((BE, NEURONS), lambda b: (b, 0)),
        out_shape=jax.ShapeDtypeStruct((N_EDGES, NEURONS), jnp.float32),
    )(edge_attr, We, be.reshape(1, NEURONS))


def _eye4():
    r = lax.broadcasted_iota(jnp.int32, (HEADS, HEADS), 0)
    c = lax.broadcasted_iota(jnp.int32, (HEADS, HEADS), 1)
    return (r == c).astype(jnp.float32)


def _ab_body(hi_ref, hj_ref, ea_ref, wt_ref, wb_ref, ai_ref, aj_ref,
             g_ref, b_ref, et_ref, sp_scr, sum_scr, sq_scr, mx_scr):
    p = pl.program_id(0)
    b = pl.program_id(1)

    @pl.when(p == 0)
    def _phase0():
        @pl.when(b == 0)
        def _init():
            sum_scr[...] = jnp.zeros((HEADS, 1), jnp.float32)
            sq_scr[...] = jnp.zeros((HEADS, 1), jnp.float32)
            mx_scr[...] = jnp.full((HEADS, 1), -1e30, jnp.float32)

        r = jnp.dot(ea_ref[...], wb_ref[...], preferred_element_type=jnp.float32)
        xi = _softplus(jnp.dot(hi_ref[:, :NEURONS], wt_ref[...],
                               preferred_element_type=jnp.float32) + r)
        xj = _softplus(jnp.dot(hj_ref[:, :NEURONS], wt_ref[...],
                               preferred_element_type=jnp.float32) + r)
        pi = xi * ai_ref[...]
        pj = xj * aj_ref[...]
        cols = []
        for h in range(HEADS):
            s = (jnp.sum(pi[:, h * NEURONS:(h + 1) * NEURONS], axis=1, keepdims=True)
                 + jnp.sum(pj[:, h * NEURONS:(h + 1) * NEURONS], axis=1, keepdims=True))
            cols.append(s)
        logit = jnp.concatenate(cols, axis=1)
        logit_t = jax.lax.dot_general(_eye4(), logit, (((1,), (1,)), ((), ())),
                                      preferred_element_type=jnp.float32)
        sp = _softplus(logit_t)
        sp_scr[0:HEADS, pl.ds(b * BE, BE)] = sp
        sum_scr[...] += jnp.sum(sp, axis=1, keepdims=True)
        sq_scr[...] += jnp.sum(sp * sp, axis=1, keepdims=True)
        mx_scr[...] = jnp.maximum(mx_scr[...],
                                  jnp.max(sp, axis=1, keepdims=True))

    @pl.when(p == 1)
    def _phase1():
        n = jnp.float32(N_EDGES)
        m = sum_scr[...] / n
        v = sq_scr[...] / n - m * m
        inv = g_ref[...] * lax.rsqrt(v + 1e-5)
        gmax = _softplus((mx_scr[...] - m) * inv + b_ref[...])
        sp = sp_scr[0:HEADS, pl.ds(b * BE, BE)]
        at = _softplus((sp - m) * inv + b_ref[...])
        et_ref[0:HEADS, :] = jnp.exp(at - gmax)
        et_ref[HEADS:8, :] = jnp.zeros((8 - HEADS, BE), jnp.float32)


def _edge_attention(HI, HJ, EA, Wt, Wb, att_i, att_j, g, bb):
    zero = lambda p, b: (0, 0)
    ph0 = lambda p, b: (jnp.where(p == 0, b, 0), 0)
    ph1 = lambda p, b: (0, jnp.where(p == 0, 0, b))
    return pl.pallas_call(
        _ab_body,
        grid=(2, NB),
        in_specs=[
            pl.BlockSpec((BE, 128), ph0),
            pl.BlockSpec((BE, 128), ph0),
            pl.BlockSpec((BE, NEURONS), ph0),
            pl.BlockSpec((NEURONS, HEADS * NEURONS), zero),
            pl.BlockSpec((NEURONS, HEADS * NEURONS), zero),
            pl.BlockSpec((1, HEADS * NEURONS), zero),
            pl.BlockSpec((1, HEADS * NEURONS), zero),
            pl.BlockSpec((HEADS, 1), zero),
            pl.BlockSpec((HEADS, 1), zero),
        ],
        out_specs=pl.BlockSpec((8, BE), ph1),
        out_shape=jax.ShapeDtypeStruct((8, N_EDGES), jnp.float32),
        scratch_shapes=[
            pltpu.VMEM((8, N_EDGES), jnp.float32),
            pltpu.VMEM((HEADS, 1), jnp.float32),
            pltpu.VMEM((HEADS, 1), jnp.float32),
            pltpu.VMEM((HEADS, 1), jnp.float32),
        ],
    )(HI, HJ, EA, Wt, Wb, att_i, att_j, g, bb)


def _c_body(hj_ref, ea_ref, et_ref, sr_ref, wt_ref, wb_ref, out_ref):
    r = jnp.dot(ea_ref[...], wb_ref[...], preferred_element_type=jnp.float32)
    xj = _softplus(jnp.dot(hj_ref[:, :NEURONS], wt_ref[...],
                           preferred_element_type=jnp.float32) + r)
    w_t = et_ref[0:HEADS, :] / (sr_ref[0:HEADS, :] + 1e-16)
    w = jax.lax.dot_general(w_t, _eye4(), (((0,), (0,)), ((), ())),
                            preferred_element_type=jnp.float32)
    acc = jnp.zeros((hj_ref.shape[0], NEURONS), jnp.float32)
    for h in range(HEADS):
        acc += xj[:, h * NEURONS:(h + 1) * NEURONS] * w[:, h:h + 1]
    out_ref[...] = jnp.concatenate(
        [acc, jnp.zeros((hj_ref.shape[0], 128 - NEURONS), jnp.float32)], 1)


def _edge_messages(HJ, EA, ET, SR, Wt, Wb):
    zero = lambda b: (0, 0)
    blk = lambda b: (b, 0)
    blk_t = lambda b: (0, b)
    return pl.pallas_call(
        _c_body,
        grid=(NB,),
        in_specs=[
            pl.BlockSpec((BE, 128), blk),
            pl.BlockSpec((BE, NEURONS), blk),
            pl.BlockSpec((8, BE), blk_t),
            pl.BlockSpec((8, BE), blk_t),
            pl.BlockSpec((NEURONS, HEADS * NEURONS), zero),
            pl.BlockSpec((NEURONS, HEADS * NEURONS), zero),
        ],
        out_specs=pl.BlockSpec((BE, 128), blk),
        out_shape=jax.ShapeDtypeStruct((N_EDGES, 128), jnp.float32),
    )(HJ, EA, ET, SR, Wt, Wb)


def _n_body(agg_ref, bias_ref, g_ref, b_ref, out_ref):
    h1 = (agg_ref[0:N_NODES, :NEURONS]
          + agg_ref[NP:NP + N_NODES, :NEURONS]) / HEADS + bias_ref[...]
    m = jnp.mean(h1, axis=0, keepdims=True)
    d = h1 - m
    v = jnp.mean(d * d, axis=0, keepdims=True)
    out = _softplus(d * lax.rsqrt(v + 1e-5) * g_ref[...] + b_ref[...])
    out_ref[...] = jnp.concatenate(
        [out, jnp.zeros((N_NODES, 128 - NEURONS), jnp.float32)], 1)


def _node_update(aggr2, bias, g, bb):
    return pl.pallas_call(
        _n_body,
        out_shape=jax.ShapeDtypeStruct((N_NODES, 128), jnp.float32),
    )(aggr2, bias.reshape(1, NEURONS), g.reshape(1, NEURONS),
      bb.reshape(1, NEURONS))


def _r_body(h_ref, batch_ref, gf_ref, w1h_ref, w1g_ref, b1_ref, w2_ref,
            b2_ref, out_ref):
    h = h_ref[:, :NEURONS]
    bvec = batch_ref[...]
    cols = lax.broadcasted_iota(jnp.int32, (1, 64), 1)
    onehot = (bvec == cols).astype(jnp.float32)
    gw = jnp.dot(gf_ref[...], w1g_ref[...],
                 preferred_element_type=jnp.float32)
    a = _softplus(jnp.dot(h, w1h_ref[...], preferred_element_type=jnp.float32)
                  + jnp.dot(onehot, gw, preferred_element_type=jnp.float32)
                  + b1_ref[...])
    a = jnp.dot(a, w2_ref[...], preferred_element_type=jnp.float32) \
        + b2_ref[...]
    amax = jnp.max(a)
    e = jnp.exp(a - amax)
    sb = jax.lax.dot_general(onehot, e, (((0,), (0,)), ((), ())))
    srow = jnp.dot(onehot, sb, preferred_element_type=jnp.float32)
    w = e / (srow + 1e-16)
    hw = h * w
    ynum = jax.lax.dot_general(onehot, hw, (((0,), (0,)), ((), ())))
    counts = jnp.sum(onehot, axis=0, keepdims=True)
    out_ref[...] = ynum / jnp.maximum(counts, 1.0).reshape(64, 1)


def _readout(h, batch, global_feat, W1, b1, W2, b2):
    return pl.pallas_call(
        _r_body,
        out_shape=jax.ShapeDtypeStruct((64, NEURONS), jnp.float32),
    )(h, batch.reshape(N_NODES, 1), global_feat,
      W1[:NEURONS], W1[NEURONS:], b1.reshape(1, 32), W2, b2.reshape(1, 1))


NC = 2
NS = 16
NW = NC * NS
CH = 128
NCH = N_EDGES // CH
NP = 10240


def _chunk_range(wid, total, nworkers):
    base = total // nworkers
    extra = total % nworkers
    start = wid * base + jnp.minimum(wid, extra)
    n = base + jnp.where(wid < extra, 1, 0)
    return start, n


def _sc_mesh():
    return plsc.VectorSubcoreMesh(core_axis_name="c", subcore_axis_name="s")


MAXW = NCH // NW + 1
NBUF = 4


def _gather_pair(h, row, col):
    @functools.partial(
        pl.kernel,
        out_type=(jax.ShapeDtypeStruct((N_EDGES, 128), jnp.float32),
                  jax.ShapeDtypeStruct((N_EDGES, 128), jnp.float32)),
        mesh=_sc_mesh(),
        scratch_types=[
            [pltpu.VMEM((CH,), jnp.int32) for _ in range(NBUF)],
            [pltpu.VMEM((CH, 128), jnp.float32) for _ in range(NBUF)],
            [pltpu.SemaphoreType.DMA for _ in range(NBUF)],
        ],
    )
    def k(h_hbm, row_hbm, col_hbm, hi_hbm, hj_hbm, idxb, bufs, semg):
        wid = lax.axis_index("s") * NC + lax.axis_index("c")
        start, n = _chunk_range(wid, NCH, NW)
        n4 = n // NBUF
        rem = n - NBUF * n4

        def do_array(idx_hbm, out_hbm):
            def gather_k(base, k_):
                for b in range(k_):
                    pltpu.sync_copy(
                        idx_hbm.at[pl.ds((start + base + b) * CH, CH)],
                        idxb[b])
                cps = [pltpu.async_copy(
                    h_hbm.at[idxb[b]], bufs[b], semg[b]) for b in range(k_)]
                for b in range(k_):
                    cps[b].wait()
                    pltpu.sync_copy(
                        bufs[b],
                        out_hbm.at[pl.ds((start + base + b) * CH, CH)])

            def grp(g, c):
                gather_k(g * NBUF, NBUF)
                return c

            lax.fori_loop(0, n4, grp, 0)

            @pl.when(rem == 3)
            def _tail():
                gather_k(n4 * NBUF, 3)

        do_array(row_hbm, hi_hbm)
        do_array(col_hbm, hj_hbm)

    return k(h, row, col)


def _attn_norm(et0, et1, et2, et3, row, zeros_n):
    rows_per_tile = NP // NS

    out_t = tuple(jax.ShapeDtypeStruct((N_EDGES,), jnp.float32)
                  for _ in range(HEADS))

    @functools.partial(
        pl.kernel,
        out_type=out_t,
        mesh=_sc_mesh(),
        scratch_types=[
            [pltpu.VMEM_SHARED((NP,), jnp.float32) for _ in range(2)],
            pltpu.VMEM((CH,), jnp.int32),
            pltpu.VMEM((CH,), jnp.float32),
            pltpu.SemaphoreType.DMA,
        ],
    )
    def k(e0, e1, e2, e3, row_hbm, z_hbm, s0, s1, s2, s3,
          shead, idx_v, val_v, sem):
        cid = lax.axis_index("c")
        sid = lax.axis_index("s")
        zoff = sid * rows_per_tile
        start, n = _chunk_range(sid, NCH, NS)

        for hh in range(2):
            pltpu.sync_copy(z_hbm.at[pl.ds(zoff, rows_per_tile)],
                            shead[hh].at[pl.ds(zoff, rows_per_tile)])
        plsc.subcore_barrier()

        def scat_pair(ins):
            def scat(i, c):
                off = (start + i) * CH
                pltpu.sync_copy(row_hbm.at[pl.ds(off, CH)], idx_v)
                for hh in range(2):
                    pltpu.sync_copy(ins[hh].at[pl.ds(off, CH)], val_v)
                    pltpu.sync_copy(val_v, shead[hh].at[idx_v], add=True)
                return c
            lax.fori_loop(0, n, scat, 0)

        def gath_pair(outs):
            def gath(i, c):
                off = (start + i) * CH
                pltpu.sync_copy(row_hbm.at[pl.ds(off, CH)], idx_v)
                for hh in range(2):
                    pltpu.async_copy(shead[hh].at[idx_v], val_v, sem).wait()
                    pltpu.sync_copy(val_v, outs[hh].at[pl.ds(off, CH)])
                return c
            lax.fori_loop(0, n, gath, 0)

        @pl.when(cid == 0)
        def _scat0():
            scat_pair([e0, e1])

        @pl.when(cid == 1)
        def _scat1():
            scat_pair([e2, e3])

        plsc.subcore_barrier()

        @pl.when(cid == 0)
        def _gath0():
            gath_pair([s0, s1])

        @pl.when(cid == 1)
        def _gath1():
            gath_pair([s2, s3])

    return k(et0, et1, et2, et3, row, zeros_n)


def _scatter_rows(values, row, zeros_tab):
    rows_per_tile = NP // NS

    @functools.partial(
        pl.kernel,
        out_type=jax.ShapeDtypeStruct((2 * NP, 128), jnp.float32),
        mesh=_sc_mesh(),
        scratch_types=[
            pltpu.VMEM_SHARED((NP, 128), jnp.float32),
            pltpu.VMEM((CH,), jnp.int32),
            pltpu.VMEM((CH, 128), jnp.float32),
        ],
    )
    def k(val_hbm, row_hbm, z_hbm, out_hbm, shared, idx_v, rows_v):
        cid = lax.axis_index("c")
        sid = lax.axis_index("s")
        wid = sid * NC + cid
        zoff = sid * rows_per_tile
        pltpu.sync_copy(z_hbm.at[pl.ds(zoff, rows_per_tile)],
                        shared.at[pl.ds(zoff, rows_per_tile)])
        plsc.subcore_barrier()

        start, n = _chunk_range(wid, NCH, NW)

        def scat(i, c):
            off = (start + i) * CH
            pltpu.sync_copy(row_hbm.at[pl.ds(off, CH)], idx_v)
            pltpu.sync_copy(val_hbm.at[pl.ds(off, CH)], rows_v)
            pltpu.sync_copy(rows_v, shared.at[idx_v], add=True)
            return c

        lax.fori_loop(0, n, scat, 0)
        plsc.subcore_barrier()
        pltpu.sync_copy(shared.at[pl.ds(zoff, rows_per_tile)],
                        out_hbm.at[pl.ds(cid * NP + zoff, rows_per_tile)])

    return k(values, row, zeros_tab)


def kernel(x, edge_index, edge_attr, batch, global_feat, cluster, Wn, bn_,
           We, be, gat_W, gat_att, gat_bias, gat_bn_g, gat_bn_b, bn_g, bn_b,
           W1, b1, W2, b2):
    pad = jnp.zeros((MAXW * CH,), jnp.int32)
    row = jnp.concatenate([edge_index[0], pad])
    col = jnp.concatenate([edge_index[1], pad])
    zeros_n = jnp.zeros((NP,), jnp.float32)
    zeros_tab = jnp.zeros((NP, 128), jnp.float32)

    h = _node_embed(x, Wn, bn_)
    ea = _edge_embed(edge_attr, We, be)

    for l in range(NL):
        Wt = gat_W[l][:NEURONS]
        Wb = gat_W[l][NEURONS:]
        att = gat_att[l]
        att_i = att[:, :NEURONS].reshape(1, HEADS * NEURONS)
        att_j = att[:, NEURONS:].reshape(1, HEADS * NEURONS)

        HI, HJ = _gather_pair(h, row, col)
        ET = _edge_attention(HI, HJ, ea, Wt, Wb, att_i, att_j,
                             gat_bn_g[l].reshape(HEADS, 1),
                             gat_bn_b[l].reshape(HEADS, 1))
        srs = _attn_norm(ET[0], ET[1], ET[2], ET[3], row, zeros_n)
        SR = jnp.concatenate(
            [s.reshape(1, N_EDGES) for s in srs]
            + [jnp.zeros((8 - HEADS, N_EDGES), jnp.float32)], 0)
        C = _edge_messages(HJ, ea, ET, SR, Wt, Wb)
        aggr2 = _scatter_rows(C, row, zeros_tab)
        h = _node_update(aggr2, gat_bias[l], bn_g[l], bn_b[l])

    return _readout(h, batch, global_feat, W1, b1, W2, b2)
